# baseline (speedup 1.0000x reference)
import functools
import math

import jax
import jax.numpy as jnp
from jax import lax
from jax.experimental import pallas as pl
from jax.experimental.pallas import tpu as pltpu
from jax.experimental.pallas import tpu_sc as plsc

N_NODES_C = 10000
N_EDGES_C = 320000
D_FEAT_C = 128
INV_SQRT_D = 1.0 / math.sqrt(D_FEAT_C)

NC = 2
NS = 16
NW = NC * NS
LANES = 16

E_PER_TILE = N_EDGES_C // NW


def _rowsum_tc_kernel(x_ref, o_ref):
    o_ref[...] = jnp.sum(x_ref[...], axis=1, keepdims=True)


def _rowsum(node_embeds):
    n = node_embeds.shape[0]
    return pl.pallas_call(
        _rowsum_tc_kernel,
        out_shape=jax.ShapeDtypeStruct((n, 1), jnp.float32),
    )(node_embeds)


def _edge_score_sc(table_hbm, src_hbm, dst_hbm, out_hbm,
                   table_v, src_v, dst_v, out_v, sem):
    wid = lax.axis_index("s") * NC + lax.axis_index("c")
    base = wid * E_PER_TILE
    ct = pltpu.make_async_copy(table_hbm, table_v, sem)
    cs = pltpu.make_async_copy(src_hbm.at[pl.ds(base, E_PER_TILE)], src_v, sem)
    cd = pltpu.make_async_copy(dst_hbm.at[pl.ds(base, E_PER_TILE)], dst_v, sem)
    ct.start()
    cs.start()
    cd.start()
    ct.wait()
    cs.wait()
    cd.wait()

    @plsc.parallel_loop(0, E_PER_TILE // LANES, step=1, unroll=8)
    def body(i):
        off = i * LANES
        si = src_v[pl.ds(off, LANES)]
        di = dst_v[pl.ds(off, LANES)]
        a = plsc.load_gather(table_v, [si])
        b = plsc.load_gather(table_v, [di])
        out_v[pl.ds(off, LANES)] = (a - b) * INV_SQRT_D

    pltpu.sync_copy(out_v, out_hbm.at[pl.ds(base, E_PER_TILE)])


@jax.jit
def kernel(node_embeds, edge_index):
    rowsum = _rowsum(node_embeds).reshape(N_NODES_C)
    idx = edge_index.astype(jnp.int32)
    src = idx[0]
    dst = idx[1]

    mesh = plsc.VectorSubcoreMesh(core_axis_name="c", subcore_axis_name="s")
    score = pl.kernel(
        _edge_score_sc,
        out_type=jax.ShapeDtypeStruct((N_EDGES_C,), jnp.float32),
        mesh=mesh,
        scratch_types=[
            pltpu.VMEM((N_NODES_C,), jnp.float32),
            pltpu.VMEM((E_PER_TILE,), jnp.int32),
            pltpu.VMEM((E_PER_TILE,), jnp.int32),
            pltpu.VMEM((E_PER_TILE,), jnp.float32),
            pltpu.SemaphoreType.DMA,
        ],
        compiler_params=pltpu.CompilerParams(
            needs_layout_passes=False,
            skip_device_barrier=True,
            disable_bounds_checks=True,
            disable_semaphore_checks=True,
        ),
    )(rowsum, src, dst)
    return score


# speedup vs baseline: 1.7991x; 1.7991x over previous
import math

import jax
import jax.numpy as jnp
from jax import lax
from jax.experimental import pallas as pl
from jax.experimental.pallas import tpu as pltpu
from jax.experimental.pallas import tpu_sc as plsc

N_NODES_C = 10000
N_EDGES_C = 320000
D_FEAT_C = 128
INV_SQRT_D = 1.0 / math.sqrt(D_FEAT_C)

NC = 2
NS = 16
NW = NC * NS
LANES = 16

E_PER_TILE = N_EDGES_C // NW


ROWSUM_BLK = 1024
ROWSUM_CHUNKS = 10


NBUF = 6


def _rowsum_tc_kernel(x_hbm, o_ref, *scratch):
    bufs, sems = scratch[:NBUF], scratch[NBUF:]

    def copy(t):
        if t == ROWSUM_CHUNKS - 1:
            rows = N_NODES_C - (ROWSUM_CHUNKS - 1) * ROWSUM_BLK
            return pltpu.make_async_copy(
                x_hbm.at[pl.ds(t * ROWSUM_BLK, rows)],
                bufs[t % NBUF].at[pl.ds(0, rows)], sems[t % NBUF])
        return pltpu.make_async_copy(
            x_hbm.at[pl.ds(t * ROWSUM_BLK, ROWSUM_BLK)],
            bufs[t % NBUF], sems[t % NBUF])

    for t in range(NBUF - 1):
        copy(t).start()
    for t in range(ROWSUM_CHUNKS):
        if t + NBUF - 1 < ROWSUM_CHUNKS:
            copy(t + NBUF - 1).start()
        copy(t).wait()
        s = jnp.sum(bufs[t % NBUF][...], axis=1)
        o_ref[pl.ds(t * 8, 8), :] = s.reshape(8, 128)


def _rowsum_2d(node_embeds):
    return pl.pallas_call(
        _rowsum_tc_kernel,
        in_specs=[pl.BlockSpec(memory_space=pltpu.MemorySpace.HBM)],
        name="rowsum_stream",
        out_shape=jax.ShapeDtypeStruct((80, 128), jnp.float32),
        scratch_shapes=(
            [pltpu.VMEM((ROWSUM_BLK, D_FEAT_C), jnp.float32)] * NBUF
            + [pltpu.SemaphoreType.DMA] * NBUF
        ),
    )(pltpu.with_memory_space_constraint(node_embeds, pltpu.MemorySpace.HBM))


WIN = 10240


def _edge_score_sc(table_hbm, ei_hbm, out_hbm,
                   table_v, ei_v, out_v, sem, sem2):
    wid = lax.axis_index("s") * NC + lax.axis_index("c")
    base = wid * E_PER_TILE
    abase = pl.multiple_of(
        jnp.minimum((base // 128) * 128, N_EDGES_C - WIN), 128)
    off0 = base - abase
    half = WIN // 2
    ct = pltpu.make_async_copy(table_hbm, table_v, sem)
    ce0 = pltpu.make_async_copy(
        ei_hbm.at[:, pl.ds(abase, half)], ei_v.at[:, pl.ds(0, half)], sem2)
    ce1 = pltpu.make_async_copy(
        ei_hbm.at[:, pl.ds(abase + half, half)],
        ei_v.at[:, pl.ds(half, half)], sem)
    ct.start()
    ce0.start()
    ce1.start()
    ct.wait()
    ce0.wait()

    n_steps = E_PER_TILE // LANES

    def gather_16(i):
        off = off0 + i * LANES
        si = ei_v[0, pl.ds(off, LANES)]
        di = ei_v[1, pl.ds(off, LANES)]
        a = plsc.load_gather(table_v, [si >> 7, si & 127])
        b = plsc.load_gather(table_v, [di >> 7, di & 127])
        out_v[pl.ds(i * LANES, LANES)] = (a - b) * INV_SQRT_D

    first = (half - 256) // LANES

    @plsc.parallel_loop(0, first, step=1, unroll=8)
    def body_a(i):
        gather_16(i)

    ce1.wait()

    @plsc.parallel_loop(first, n_steps, step=1, unroll=8)
    def body_b(i):
        gather_16(i)

    pltpu.sync_copy(out_v, out_hbm.at[pl.ds(base, E_PER_TILE)])


@jax.jit
def kernel(node_embeds, edge_index):
    rowsum = _rowsum_2d(node_embeds)
    idx = edge_index.astype(jnp.int32)

    mesh = plsc.VectorSubcoreMesh(core_axis_name="c", subcore_axis_name="s")
    score = pl.kernel(
        _edge_score_sc,
        out_type=jax.ShapeDtypeStruct((N_EDGES_C,), jnp.float32),
        mesh=mesh,
        scratch_types=[
            pltpu.VMEM((80, 128), jnp.float32),
            pltpu.VMEM((2, WIN), jnp.int32),
            pltpu.VMEM((E_PER_TILE,), jnp.float32),
            pltpu.SemaphoreType.DMA,
            pltpu.SemaphoreType.DMA,
        ],
        compiler_params=pltpu.CompilerParams(needs_layout_passes=False),
    )(rowsum, idx)
    return score
